# baseline (speedup 1.0000x reference)
import functools

import jax
import jax.numpy as jnp
from jax.experimental import pallas as pl
from jax.experimental.pallas import tpu as pltpu

_MIN_DISABLED = 32
_MAX_DISABLED = 128
_LANE = 128


@functools.cache
def _disabled_span(tof_count: int) -> tuple[int, int]:
    with jax.ensure_compile_time_eval():
        key = jax.random.key(42)
        k1, k2 = jax.random.split(key)
        count = int(jax.random.randint(k1, (), _MIN_DISABLED, _MAX_DISABLED + 1))
        start = int(jax.random.randint(k2, (), 0, tof_count))
    return start, count


def _masked_copy_body(img_ref, out_ref, *rest, kept_runs, masked_tiles,
                      start, count, tof_count):
    n_masked = len(masked_tiles)
    scratches = rest[:n_masked]
    sems = rest[n_masked:]
    sem_i = 0

    big_copies = []
    for c0, c1 in kept_runs:
        cp = pltpu.make_async_copy(
            img_ref.at[:, c0:c1], out_ref.at[:, c0:c1], sems[sem_i])
        sem_i += 1
        cp.start()
        big_copies.append(cp)

    in_copies = []
    for j, t in enumerate(masked_tiles):
        cp = pltpu.make_async_copy(
            img_ref.at[:, t * _LANE:(t + 1) * _LANE], scratches[j], sems[sem_i])
        sem_i += 1
        cp.start()
        in_copies.append(cp)

    out_copies = []
    for j, t in enumerate(masked_tiles):
        in_copies[j].wait()
        cols = t * _LANE + jax.lax.broadcasted_iota(
            jnp.int32, scratches[j].shape, 1)
        disabled = ((cols - start) % tof_count) < count
        scratches[j][...] = jnp.where(disabled, jnp.float32(0.0),
                                      scratches[j][...])
        cp = pltpu.make_async_copy(
            scratches[j], out_ref.at[:, t * _LANE:(t + 1) * _LANE], sems[sem_i])
        sem_i += 1
        cp.start()
        out_copies.append(cp)

    for cp in big_copies + out_copies:
        cp.wait()


def kernel(img):
    rows, tof_count = img.shape
    start, count = _disabled_span(tof_count)
    end = start + count

    n_tiles = tof_count // _LANE
    t0 = start // _LANE
    t1 = ((end - 1) // _LANE) % n_tiles
    masked_tiles = sorted({t0, t1})

    kept_runs = []
    run_start = None
    for t in range(n_tiles):
        if t in masked_tiles:
            if run_start is not None:
                kept_runs.append((run_start * _LANE, t * _LANE))
                run_start = None
        elif run_start is None:
            run_start = t
    if run_start is not None:
        kept_runs.append((run_start * _LANE, n_tiles * _LANE))

    n_copies = len(kept_runs) + 2 * len(masked_tiles)
    body = functools.partial(
        _masked_copy_body, kept_runs=kept_runs, masked_tiles=masked_tiles,
        start=start, count=count, tof_count=tof_count)
    return pl.pallas_call(
        body,
        in_specs=[pl.BlockSpec(memory_space=pl.ANY)],
        out_specs=pl.BlockSpec(memory_space=pl.ANY),
        out_shape=jax.ShapeDtypeStruct((rows, tof_count), jnp.float32),
        scratch_shapes=(
            [pltpu.VMEM((rows, _LANE), jnp.float32) for _ in masked_tiles]
            + [pltpu.SemaphoreType.DMA for _ in range(n_copies)]
        ),
    )(img)
